# baseline (speedup 1.0000x reference)
import functools

import jax
import jax.numpy as jnp
from jax import lax
from jax.experimental import pallas as pl
from jax.experimental.pallas import tpu as pltpu
from jax.experimental.pallas import tpu_sc as plsc

SEQ_LEN = 2048
START_TOK = 0
END_TOK = 2
PAD_TOK = 1
LANES = 16


def _trim_budgets(L1, L2, budget):
    if L1 + L2 <= budget:
        return L1, L2
    k1 = min(L1, max((budget + 1) // 2, budget - L2))
    k2 = min(L2, max(budget // 2, budget - L1))
    return max(k1, 0), max(k2, 0)


@functools.cache
def _build_packer(B, L1, L2):
    budget = SEQ_LEN - 4
    k1, k2 = _trim_budgets(L1, L2, budget)
    half = SEQ_LEN // 2
    assert k1 == half - 2 and k2 == half - 2 and 4 + k1 + k2 == SEQ_LEN
    assert L1 >= half and L2 >= half and B == 16
    nvec = half // LANES

    mesh = plsc.VectorSubcoreMesh(core_axis_name="c", subcore_axis_name="s")

    rot_dnums = lax.GatherDimensionNumbers(
        offset_dims=(), collapsed_slice_dims=(0,), start_index_map=(0,))

    @functools.partial(
        pl.kernel,
        out_type=jax.ShapeDtypeStruct((2 * B, half), jnp.int32),
        mesh=mesh,
        scratch_types=[
            pltpu.VMEM((half,), jnp.int32),
        ],
        compiler_params=pltpu.CompilerParams(
            disable_bounds_checks=True,
            skip_device_barrier=True,
        ),
    )
    def pack(seg1_hbm, seg2_hbm, out_hbm, block):
        row = lax.axis_index("s")
        half_id = lax.axis_index("c")

        @pl.when(half_id == 0)
        def _():
            pltpu.sync_copy(seg1_hbm.at[row, pl.ds(0, half)], block)

        @pl.when(half_id == 1)
        def _():
            pltpu.sync_copy(seg2_hbm.at[row, pl.ds(0, half)], block)

        first_tok = jnp.where(half_id == 0, START_TOK, END_TOK).astype(jnp.int32)
        lane = lax.iota(jnp.int32, LANES)

        rot_idx = (lane + LANES - 1) % LANES

        def rot1(v):
            return lax.gather(
                v, rot_idx[:, None], rot_dnums, (1,),
                mode=lax.GatherScatterMode.PROMISE_IN_BOUNDS)

        prev_rot = jnp.full((LANES,), first_tok, jnp.int32)
        for j in range(nvec):
            cur = block[pl.ds(j * LANES, LANES)]
            rotc = rot1(cur)
            v = jnp.where(lane == 0, prev_rot, rotc)
            block[pl.ds(j * LANES, LANES)] = v
            prev_rot = rotc

        last = block[pl.ds(half - LANES, LANES)]
        block[pl.ds(half - LANES, LANES)] = jnp.where(
            lane == LANES - 1, jnp.int32(END_TOK), last)

        pltpu.sync_copy(block, out_hbm.at[2 * row + half_id])

    return pack


def kernel(segment_1, segment_2):
    B, L1 = segment_1.shape
    L2 = segment_2.shape[1]
    packed = _build_packer(B, L1, L2)(segment_1, segment_2)
    return packed.reshape(B, SEQ_LEN)


# speedup vs baseline: 12.6449x; 12.6449x over previous
import functools

import jax
import jax.numpy as jnp
from jax.experimental import pallas as pl

SEQ_LEN = 2048
START_TOK = 0
END_TOK = 2
PAD_TOK = 1


def _trim_budgets(L1, L2, budget):
    if L1 + L2 <= budget:
        return L1, L2
    k1 = min(L1, max((budget + 1) // 2, budget - L2))
    k2 = min(L2, max(budget // 2, budget - L1))
    return max(k1, 0), max(k2, 0)


@functools.cache
def _build_packer(B, L1, L2):
    budget = SEQ_LEN - 4
    k1, k2 = _trim_budgets(L1, L2, budget)
    pad = SEQ_LEN - (4 + k1 + k2)
    assert pad == 0

    def body(s1_ref, s2_ref, o_ref):
        s1 = s1_ref[:, :k1]
        s2 = s2_ref[:, :k2]
        start = jnp.full((B, 1), START_TOK, jnp.int32)
        split = jnp.full((B, 2), END_TOK, jnp.int32)
        end = jnp.full((B, 1), END_TOK, jnp.int32)
        o_ref[...] = jnp.concatenate([start, s1, split, s2, end], axis=1)

    return pl.pallas_call(
        body,
        out_shape=jax.ShapeDtypeStruct((B, SEQ_LEN), jnp.int32),
    )


def kernel(segment_1, segment_2):
    B, L1 = segment_1.shape
    L2 = segment_2.shape[1]
    return _build_packer(B, L1, L2)(segment_1, segment_2)


# speedup vs baseline: 12.9050x; 1.0206x over previous
import functools

import jax
import jax.numpy as jnp
from jax.experimental import pallas as pl

SEQ_LEN = 2048
START_TOK = 0
END_TOK = 2
PAD_TOK = 1


def _trim_budgets(L1, L2, budget):
    if L1 + L2 <= budget:
        return L1, L2
    k1 = min(L1, max((budget + 1) // 2, budget - L2))
    k2 = min(L2, max(budget // 2, budget - L1))
    return max(k1, 0), max(k2, 0)


@functools.cache
def _build_packer(B, L1, L2):
    budget = SEQ_LEN - 4
    k1, k2 = _trim_budgets(L1, L2, budget)
    pad = SEQ_LEN - (4 + k1 + k2)
    assert pad == 0

    w1 = -(-k1 // 128) * 128
    w2 = -(-k2 // 128) * 128

    def body(s1_ref, s2_ref, o_ref):
        s1 = s1_ref[:, :k1]
        s2 = s2_ref[:, :k2]
        start = jnp.full((B, 1), START_TOK, jnp.int32)
        split = jnp.full((B, 2), END_TOK, jnp.int32)
        end = jnp.full((B, 1), END_TOK, jnp.int32)
        o_ref[...] = jnp.concatenate([start, s1, split, s2, end], axis=1)

    return pl.pallas_call(
        body,
        grid=(1,),
        in_specs=[
            pl.BlockSpec((B, w1), lambda i: (0, 0)),
            pl.BlockSpec((B, w2), lambda i: (0, 0)),
        ],
        out_specs=pl.BlockSpec((B, SEQ_LEN), lambda i: (0, 0)),
        out_shape=jax.ShapeDtypeStruct((B, SEQ_LEN), jnp.int32),
    )


def kernel(segment_1, segment_2):
    B, L1 = segment_1.shape
    L2 = segment_2.shape[1]
    return _build_packer(B, L1, L2)(segment_1, segment_2)
